# Initial kernel scaffold; baseline (speedup 1.0000x reference)
import jax
import jax.numpy as jnp
from jax.experimental import pallas as pl


def kernel(rgb_patches, nir_patches, tir_patches, rgb_global, nir_global, tir_global, Wq_w, Wq_b, Wk_w, Wk_b):
    raise NotImplementedError("write your pallas kernel here")


# speedup vs baseline: 1.6873x; 1.6873x over previous
import functools
import math

import jax
import jax.numpy as jnp
from jax import lax
from jax.experimental import pallas as pl
from jax.experimental.pallas import tpu as pltpu
from jax.experimental.pallas import tpu_sc as plsc

DIM = 768
NPATCH = 576
BATCH = 32
K1 = 112
K2 = 224
NROWS = 192
ROWW = 1152
NWORKERS = 32
ROWS_PER_W = NROWS // NWORKERS
CHUNKS = ROWW // 16


def _projq_body(g_ref, wq_ref, bq_ref, q_ref):
    q_ref[...] = lax.dot_general(
        g_ref[...], wq_ref[...], (((1,), (1,)), ((), ())),
        preferred_element_type=jnp.float32) + bq_ref[...]


def _proj_q(gflat, wq, bq):
    return pl.pallas_call(
        _projq_body,
        out_shape=jax.ShapeDtypeStruct((3 * BATCH, DIM), jnp.float32),
    )(gflat, wq, bq.reshape(1, DIM))


MB = 4


def _logits_body(p_ref, wk_ref, bk_ref, c_ref, lt_ref, it_ref):
    pm = p_ref[...].reshape(MB * NPATCH, DIM)
    k_ = lax.dot_general(pm, wk_ref[...], (((1,), (1,)), ((), ())),
                         preferred_element_type=jnp.float32) + bk_ref[...]
    for i in range(MB):
        c = c_ref[i]
        ks = k_[i * NPATCH:(i + 1) * NPATCH]
        ps = pm[i * NPATCH:(i + 1) * NPATCH]
        lt_ref[i] = lax.dot_general(c, ks, (((1,), (1,)), ((), ())),
                                    preferred_element_type=jnp.float32)
        it_ref[i] = lax.dot_general(c, ps, (((1,), (1,)), ((), ())),
                                    preferred_element_type=jnp.float32)


def _logits(patches, wk, bk, c):
    return pl.pallas_call(
        _logits_body,
        grid=(BATCH // MB,),
        in_specs=[
            pl.BlockSpec((MB, NPATCH, DIM), lambda b: (b, 0, 0)),
            pl.BlockSpec((DIM, DIM), lambda b: (0, 0)),
            pl.BlockSpec((1, DIM), lambda b: (0, 0)),
            pl.BlockSpec((MB, 8, DIM), lambda b: (b, 0, 0)),
        ],
        out_specs=[
            pl.BlockSpec((MB, 8, NPATCH), lambda b: (b, 0, 0)),
            pl.BlockSpec((MB, 8, NPATCH), lambda b: (b, 0, 0)),
        ],
        out_shape=[
            jax.ShapeDtypeStruct((BATCH, 8, NPATCH), jnp.float32),
            jax.ShapeDtypeStruct((BATCH, 8, NPATCH), jnp.float32),
        ],
    )(patches, wk, bk.reshape(1, DIM), c)


def _sc_topk_body(sel_hbm, mask_hbm, u_v, buf_v, cand_v, sem):
    del sem
    wid = lax.axis_index("s") * 2 + lax.axis_index("c")
    f16_1 = jnp.full((16,), 1.0, jnp.float32)
    f16_0 = jnp.zeros((16,), jnp.float32)
    i16_1 = jnp.full((16,), 1, jnp.int32)
    i16_0 = jnp.zeros((16,), jnp.int32)
    iota16 = lax.iota(jnp.int32, 16)

    def do_row(r, rcarry):
        row = r * NWORKERS + wid
        is_inter = row < 96
        k = jnp.where(is_inter, jnp.int32(K2), jnp.int32(K1))
        nch = jnp.where(is_inter, jnp.int32(CHUNKS), jnp.int32(CHUNKS // 2))
        pltpu.sync_copy(sel_hbm.at[row], buf_v)

        def conv(j, carry):
            x = buf_v[pl.ds(j * 16, 16)]
            ub = lax.bitcast_convert_type(x, jnp.uint32)
            neg = ub >= jnp.uint32(0x80000000)
            u = jnp.where(neg, ub ^ jnp.uint32(0xFFFFFFFF),
                          ub | jnp.uint32(0x80000000))
            u_v[pl.ds(j * 16, 16)] = u
            return carry
        lax.fori_loop(0, nch, conv, jnp.int32(0))

        def count_ge(c):
            cb = jnp.broadcast_to(c, (16,))

            def step(j, acc):
                u = u_v[pl.ds(j * 16, 16)]
                return acc + jnp.where(u >= cb, i16_1, i16_0)
            return jnp.sum(lax.fori_loop(0, nch, step, i16_0))

        p = jnp.uint32(0)
        for t in range(16):
            c = p | (jnp.uint32(0x80000000) >> t)
            p = jnp.where(count_ge(c) >= k, c, p)

        nxt = p + jnp.uint32(0x00010000)
        topgt = jnp.where(nxt == jnp.uint32(0),
                          jnp.int32(0), count_ge(nxt))

        pfx = jnp.broadcast_to(p, (16,))
        m16 = jnp.full((16,), jnp.uint32(0xFFFF0000), jnp.uint32)

        def compact(j, off):
            u = u_v[pl.ds(j * 16, 16)]
            m = (u & m16) == pfx
            mi = jnp.where(m, i16_1, i16_0)
            pre = plsc.cumsum(mi)
            idx = jnp.broadcast_to(off, (16,)) + pre - 1
            plsc.store_scatter(cand_v, [idx],
                               plsc.bitcast(u, jnp.int32), mask=m)
            return off + jnp.sum(mi)
        nc = lax.fori_loop(0, nch, compact, jnp.int32(0))
        ncch = (nc + 15) // 16
        idxz = jnp.broadcast_to(nc, (16,)) + iota16
        plsc.store_scatter(cand_v, [idxz], i16_0,
                           mask=idxz < jnp.broadcast_to(ncch * 16, (16,)))

        for t in range(16, 32):
            c = p | (jnp.uint32(0x80000000) >> t)
            cb = jnp.broadcast_to(c, (16,))

            def step(j, acc, cb=cb):
                uc = plsc.bitcast(cand_v[pl.ds(j * 16, 16)], jnp.uint32)
                return acc + jnp.where(uc >= cb, i16_1, i16_0)
            cnt = topgt + jnp.sum(lax.fori_loop(0, ncch, step, i16_0))
            p = jnp.where(cnt >= k, c, p)

        pb = jnp.broadcast_to(p, (16,))
        cge = count_ge(p)
        def eqcnt(j, acc):
            u = u_v[pl.ds(j * 16, 16)]
            return acc + jnp.where(u == pb, i16_1, i16_0)
        neq = jnp.sum(lax.fori_loop(0, nch, eqcnt, i16_0))
        rem = k - (cge - neq)
        remv = jnp.broadcast_to(rem, (16,))

        def emit(j, carry):
            u = u_v[pl.ds(j * 16, 16)]
            eq = u == pb
            eqi = jnp.where(eq, i16_1, i16_0)
            pre = plsc.cumsum(eqi) + jnp.broadcast_to(carry, (16,))
            sel = (u > pb) | (eq & (pre <= remv))
            buf_v[pl.ds(j * 16, 16)] = jnp.where(sel, f16_1, f16_0)
            return carry + jnp.sum(eqi)
        lax.fori_loop(0, nch, emit, jnp.int32(0))

        def zpad(j, carry):
            buf_v[pl.ds(j * 16, 16)] = f16_0
            return carry
        lax.fori_loop(nch, CHUNKS, zpad, jnp.int32(0))
        pltpu.sync_copy(buf_v, mask_hbm.at[row])
        return rcarry

    lax.fori_loop(0, ROWS_PER_W, do_row, jnp.int32(0))


def _sc_topk(sel):
    mesh = plsc.VectorSubcoreMesh(core_axis_name="c", subcore_axis_name="s")
    fn = functools.partial(
        pl.kernel,
        out_type=jax.ShapeDtypeStruct((NROWS, ROWW), jnp.float32),
        mesh=mesh,
        compiler_params=pltpu.CompilerParams(needs_layout_passes=False),
        scratch_types=[
            pltpu.VMEM((ROWW,), jnp.uint32),
            pltpu.VMEM((ROWW,), jnp.float32),
            pltpu.VMEM((ROWW + 16,), jnp.int32),
            pltpu.SemaphoreType.DMA,
        ],
    )(_sc_topk_body)
    return fn(sel)


def _apply_body(p_ref, m1_ref, m2_ref, m3_ref, o_ref, c1, c2, c3):
    for i in range(MB):
        s = (m1_ref[i, 0, c1:c1 + NPATCH] + m2_ref[i, 0, c2:c2 + NPATCH]
             + m3_ref[i, 0, c3:c3 + NPATCH])
        maskv = (s > 0.0).astype(jnp.float32)
        o_ref[i] = p_ref[i] * maskv[:, None]


def _apply(patches, maskf3, srcs):
    (r1, c1), (r2, c2), (r3, c3) = srcs

    def mspec(rowbase):
        return pl.BlockSpec((MB, 1, ROWW),
                            lambda b, rb=rowbase: (rb // MB + b, 0, 0))

    body = functools.partial(_apply_body, c1=c1, c2=c2, c3=c3)
    return pl.pallas_call(
        body,
        grid=(BATCH // MB,),
        in_specs=[
            pl.BlockSpec((MB, NPATCH, DIM), lambda b: (b, 0, 0)),
            mspec(r1), mspec(r2), mspec(r3),
        ],
        out_specs=pl.BlockSpec((MB, NPATCH, DIM), lambda b: (b, 0, 0)),
        out_shape=jax.ShapeDtypeStruct((BATCH, NPATCH, DIM), jnp.float32),
    )(patches, maskf3, maskf3, maskf3)


def kernel(rgb_patches, nir_patches, tir_patches, rgb_global, nir_global,
           tir_global, Wq_w, Wq_b, Wk_w, Wk_b):
    G = jnp.stack([rgb_global, nir_global, tir_global], axis=1)
    Q = _proj_q(G.reshape(3 * BATCH, DIM), Wq_w, Wq_b)
    C = jnp.concatenate(
        [Q.reshape(BATCH, 3, DIM), G,
         jnp.zeros((BATCH, 2, DIM), jnp.float32)], axis=1)

    LT_rgb, IT_rgb = _logits(rgb_patches, Wk_w, Wk_b, C)
    LT_nir, IT_nir = _logits(nir_patches, Wk_w, Wk_b, C)
    LT_tir, IT_tir = _logits(tir_patches, Wk_w, Wk_b, C)

    neg = jnp.full((BATCH, NPATCH), -jnp.inf, jnp.float32)
    sel = jnp.concatenate([
        jnp.concatenate([LT_nir[:, 0, :], LT_tir[:, 0, :]], axis=1),
        jnp.concatenate([LT_rgb[:, 1, :], LT_tir[:, 1, :]], axis=1),
        jnp.concatenate([LT_rgb[:, 2, :], LT_nir[:, 2, :]], axis=1),
        jnp.concatenate([IT_rgb[:, 3, :], neg], axis=1),
        jnp.concatenate([IT_nir[:, 4, :], neg], axis=1),
        jnp.concatenate([IT_tir[:, 5, :], neg], axis=1),
    ], axis=0)

    maskf3 = _sc_topk(sel).reshape(NROWS, 1, ROWW)

    return (_apply(rgb_patches, maskf3, ((32, 0), (64, 0), (96, 0))),
            _apply(nir_patches, maskf3, ((0, 0), (64, NPATCH), (128, 0))),
            _apply(tir_patches, maskf3, ((0, NPATCH), (32, NPATCH), (160, 0))))
